# Initial kernel scaffold; baseline (speedup 1.0000x reference)
import jax
import jax.numpy as jnp
from jax.experimental import pallas as pl


def kernel(x, edge_index, W1, b1, W2, b2, W3, b3):
    raise NotImplementedError("write your pallas kernel here")


# speedup vs baseline: 8.6596x; 8.6596x over previous
import functools

import jax
import jax.numpy as jnp
from jax import lax
from jax.experimental import pallas as pl
from jax.experimental.pallas import tpu as pltpu
from jax.experimental.pallas import tpu_sc as plsc

N = 10000
E = 320000
NC, NS, L = 2, 16, 16
NW = NC * NS
CHUNK = 128
CHUNKS = 80
EPAD = NW * CHUNKS * CHUNK
OFF = 624
ROWS = 640

_MESH = dict(core_axis_name="c", subcore_axis_name="s")


PCHUNK = 64
PCHUNKS = EPAD // (NW * PCHUNK)


@functools.partial(
    pl.kernel,
    out_type=jax.ShapeDtypeStruct((NC, N, 128), jnp.float32),
    mesh=plsc.VectorSubcoreMesh(**_MESH),
    scratch_types=[
        pltpu.VMEM((PCHUNKS, PCHUNK), jnp.int32),
        pltpu.VMEM((PCHUNK, 128), jnp.float32),
        pltpu.VMEM_SHARED((N + L, 128), jnp.float32),
    ],
)
def _deg_kernel(dstp_hbm, deg_hbm, dst_v, ones_v, deg_sh):
    c = lax.axis_index("c")
    s = lax.axis_index("s")
    w = s * NC + c

    pltpu.sync_copy(dstp_hbm.at[w], dst_v)

    @pl.loop(0, PCHUNK)
    def _fill_zeros(i):
        for k in range(128 // L):
            ones_v[i, pl.ds(k * L, L)] = jnp.zeros((L,), jnp.float32)

    for off in range(0, ROWS, PCHUNK):
        pltpu.sync_copy(ones_v, deg_sh.at[pl.ds(s * OFF + off, PCHUNK)])

    @pl.loop(0, PCHUNK)
    def _fill_ones(i):
        for k in range(128 // L):
            ones_v[i, pl.ds(k * L, L)] = jnp.full((L,), 1.0, jnp.float32)

    plsc.subcore_barrier()

    @pl.loop(0, PCHUNKS)
    def _scatter(j):
        pltpu.sync_copy(ones_v, deg_sh.at[dst_v.at[j]], add=True)

    plsc.subcore_barrier()
    pltpu.sync_copy(deg_sh.at[pl.ds(s * OFF, ROWS)],
                    deg_hbm.at[c, pl.ds(s * OFF, ROWS)])


def _make_propagate(D):
    @functools.partial(
        pl.kernel,
        out_type=jax.ShapeDtypeStruct((NC, N, D), jnp.float32),
        mesh=plsc.VectorSubcoreMesh(**_MESH),
        scratch_types=[
            pltpu.VMEM((PCHUNKS * PCHUNK,), jnp.int32),
            pltpu.VMEM((PCHUNKS, PCHUNK), jnp.int32),
            pltpu.VMEM((2, PCHUNK, D), jnp.float32),
            pltpu.VMEM_SHARED((N + L, D), jnp.float32),
            pltpu.SemaphoreType.DMA,
            pltpu.SemaphoreType.DMA,
        ],
    )
    def _prop(g_hbm, srcp_hbm, dstp_hbm, acc_hbm,
              src_v, dst_v, bufs, acc_sh, sem0, sem1):
        c = lax.axis_index("c")
        s = lax.axis_index("s")
        w = s * NC + c
        sems = (sem0, sem1)

        pltpu.sync_copy(srcp_hbm.at[w], src_v)
        pltpu.sync_copy(dstp_hbm.at[w], dst_v)

        @pl.loop(0, PCHUNK)
        def _fill_zeros(i):
            for k in range(D // L):
                bufs[0, i, pl.ds(k * L, L)] = jnp.zeros((L,), jnp.float32)

        for off in range(0, ROWS, PCHUNK):
            pltpu.sync_copy(bufs.at[0], acc_sh.at[pl.ds(s * OFF + off, PCHUNK)])
        plsc.subcore_barrier()

        for b in range(2):
            pltpu.async_copy(g_hbm.at[src_v.at[pl.ds(b * PCHUNK, PCHUNK)]],
                             bufs.at[b], sems[b])

        @pl.loop(0, PCHUNKS, step=2)
        def _edges(j0):
            for b in range(2):
                j = j0 + b
                pltpu.make_async_copy(
                    g_hbm.at[src_v.at[pl.ds(0, PCHUNK)]],
                    bufs.at[b], sems[b]).wait()
                pltpu.sync_copy(bufs.at[b], acc_sh.at[dst_v.at[j]], add=True)

                @pl.when(j + 2 < PCHUNKS)
                def _prefetch():
                    pltpu.async_copy(
                        g_hbm.at[src_v.at[pl.ds((j + 2) * PCHUNK, PCHUNK)]],
                        bufs.at[b], sems[b])

        plsc.subcore_barrier()
        pltpu.sync_copy(acc_sh.at[pl.ds(s * OFF, ROWS)],
                        acc_hbm.at[c, pl.ds(s * OFF, ROWS)])

    return _prop


_prop128 = _make_propagate(128)


BR = 1000


def _row_spec(d):
    return pl.BlockSpec((BR, d), lambda i: (i, 0))


def _full_spec(r, d):
    return pl.BlockSpec((r, d), lambda i: (0, 0))


def _dinv(d0_ref, d1_ref):
    deg = d0_ref[:, 0:1] + d1_ref[:, 0:1] + 1.0
    return lax.rsqrt(deg)


def _mm1_body(x_ref, w_ref, d0_ref, d1_ref, g_ref):
    g_ref[...] = _dinv(d0_ref, d1_ref) * jnp.dot(
        x_ref[...], w_ref[...], preferred_element_type=jnp.float32,
        precision=lax.Precision.HIGHEST)


def _mm1(x, w, d0, d1):
    din, dout = w.shape
    return pl.pallas_call(
        _mm1_body,
        grid=(N // BR,),
        in_specs=[_row_spec(din), _full_spec(din, dout),
                  _row_spec(128), _row_spec(128)],
        out_specs=_row_spec(dout),
        out_shape=jax.ShapeDtypeStruct((N, dout), jnp.float32),
    )(x, w, d0, d1)


def _mid_body(a0_ref, a1_ref, gp_ref, b_ref, w_ref, d0_ref, d1_ref, gn_ref):
    dinv = _dinv(d0_ref, d1_ref)
    h = dinv * (a0_ref[...] + a1_ref[...] + gp_ref[...]) + b_ref[...]
    h = jnp.maximum(h, 0.0)
    gn_ref[...] = dinv * jnp.dot(h, w_ref[...],
                                 preferred_element_type=jnp.float32,
                                 precision=lax.Precision.HIGHEST)


def _mid(a0, a1, gp, b, w, d0, d1):
    din, dout = w.shape
    return pl.pallas_call(
        _mid_body,
        grid=(N // BR,),
        in_specs=[_row_spec(din), _row_spec(din), _row_spec(din),
                  _full_spec(1, din), _full_spec(din, dout),
                  _row_spec(128), _row_spec(128)],
        out_specs=_row_spec(dout),
        out_shape=jax.ShapeDtypeStruct((N, dout), jnp.float32),
    )(a0, a1, gp, b, w, d0, d1)


def _final_body(a0_ref, a1_ref, gp_ref, b_ref, d0_ref, d1_ref, o_ref):
    dinv = _dinv(d0_ref, d1_ref)
    s = a0_ref[:, :64] + a1_ref[:, :64] + gp_ref[:, :64]
    o_ref[...] = dinv * s + b_ref[...]


def _final(a0, a1, gp, b, d0, d1):
    d = gp.shape[1]
    return pl.pallas_call(
        _final_body,
        grid=(N // BR,),
        in_specs=[_row_spec(d), _row_spec(d), _row_spec(d),
                  _full_spec(1, 64), _row_spec(128), _row_spec(128)],
        out_specs=_row_spec(64),
        out_shape=jax.ShapeDtypeStruct((N, 64), jnp.float32),
    )(a0, a1, gp, b, d0, d1)


def kernel(x, edge_index, W1, b1, W2, b2, W3, b3):
    src = edge_index[0].astype(jnp.int32)
    dst = edge_index[1].astype(jnp.int32)
    pad = EPAD - E
    srcp = jnp.concatenate([src, jnp.zeros((pad,), jnp.int32)])
    srcp = srcp.reshape(NW, CHUNKS * CHUNK)
    dstp = jnp.concatenate([dst, jnp.full((pad,), N, jnp.int32)])
    dstp = dstp.reshape(NW, CHUNKS, CHUNK)

    dstp64 = dstp.reshape(NW, PCHUNKS, PCHUNK)

    deg2 = _deg_kernel(dstp64)
    d0, d1 = deg2[0], deg2[1]

    b1r = b1.reshape(1, -1)
    b2r = b2.reshape(1, -1)
    b3r = b3.reshape(1, -1)

    g1 = _mm1(x, W1, d0, d1)
    acc = _prop128(g1, srcp, dstp64)
    g2 = _mid(acc[0], acc[1], g1, b1r, W2, d0, d1)
    acc = _prop128(g2, srcp, dstp64)
    W3p = jnp.pad(W3, ((0, 0), (0, 64)))
    g3 = _mid(acc[0], acc[1], g2, b2r, W3p, d0, d1)
    acc3 = _prop128(g3, srcp, dstp64)
    return _final(acc3[0], acc3[1], g3, b3r, d0, d1)
